# baseline (speedup 1.0000x reference)
import functools

import jax
import jax.numpy as jnp
from jax import lax
from jax.experimental import pallas as pl
from jax.experimental.pallas import tpu as pltpu
from jax.experimental.pallas import tpu_sc as plsc


def _mean_body(x_ref, o_ref, *, inv_w):
    o_ref[...] = jnp.sum(x_ref[...], axis=1) * inv_w


@functools.partial(jax.jit, static_argnames=("block_n",))
def _time_mean(final_emb, block_n=400):
    V, W, D = final_emb.shape
    grid = (V // block_n,)
    return pl.pallas_call(
        functools.partial(_mean_body, inv_w=1.0 / W),
        grid=grid,
        in_specs=[pl.BlockSpec((block_n, W, D), lambda i: (i, 0, 0))],
        out_specs=pl.BlockSpec((block_n, D), lambda i: (i, 0)),
        out_shape=jax.ShapeDtypeStruct((V, D), jnp.float32),
    )(final_emb)


_GCHUNK = 64


def _make_sc_gather(V, D, B):
    info = plsc.get_sparse_core_info()
    NC, NS = info.num_cores, info.num_subcores
    NW = NC * NS
    b_per_w = B // NW
    CHUNK = _GCHUNK
    n_chunks = b_per_w // CHUNK
    DEPTH = 4
    mesh = plsc.VectorSubcoreMesh(core_axis_name="c", subcore_axis_name="s")

    @functools.partial(
        pl.kernel,
        mesh=mesh,
        out_type=jax.ShapeDtypeStruct((B, D), jnp.float32),
        scratch_types=[
            pltpu.VMEM((n_chunks, CHUNK), jnp.int32),
            pltpu.VMEM((b_per_w, D), jnp.float32),
            pltpu.SemaphoreType.DMA,
            pltpu.SemaphoreType.DMA,
            pltpu.SemaphoreType.DMA,
            pltpu.SemaphoreType.DMA,
            pltpu.SemaphoreType.DMA,
        ],
    )
    def gather(table_hbm, idx_hbm, out_hbm, idx_v, rows_v,
               g0, g1, g2, g3, wsem):
        wid = lax.axis_index("s") * NC + lax.axis_index("c")
        gsems = (g0, g1, g2, g3)
        pltpu.sync_copy(idx_hbm.at[pl.ds(wid * n_chunks, n_chunks)], idx_v)

        def gather_copy(j):
            return pltpu.make_async_copy(
                table_hbm.at[idx_v.at[j]],
                rows_v.at[pl.ds(j * CHUNK, CHUNK)],
                gsems[j % DEPTH],
            )

        for j in range(DEPTH):
            gather_copy(j).start()
        writes = []
        for j in range(n_chunks):
            gather_copy(j).wait()
            if j + DEPTH < n_chunks:
                gather_copy(j + DEPTH).start()
            writes.append(pltpu.async_copy(
                rows_v.at[pl.ds(j * CHUNK, CHUNK)],
                out_hbm.at[pl.ds(wid * b_per_w + j * CHUNK, CHUNK)],
                wsem,
            ))
        for wr in writes:
            wr.wait()

    return gather


def kernel(final_emb, node, time):
    V, W, D = final_emb.shape
    B = node.shape[0]
    table = _time_mean(final_emb)
    idx = node.reshape(B // _GCHUNK, _GCHUNK).astype(jnp.int32)
    rows = _make_sc_gather(V, D, B)(table, idx)
    return rows.reshape(B, 1, D)


# speedup vs baseline: 1.0605x; 1.0605x over previous
import functools

import jax
import jax.numpy as jnp
from jax import lax
from jax.experimental import pallas as pl
from jax.experimental.pallas import tpu as pltpu
from jax.experimental.pallas import tpu_sc as plsc


def _mean_body(x_ref, o_ref, *, inv_w):
    o_ref[...] = jnp.sum(x_ref[...], axis=1) * inv_w


@functools.partial(jax.jit, static_argnames=("block_n",))
def _time_mean(final_emb, block_n=400):
    V, W, D = final_emb.shape
    grid = (V // block_n,)
    return pl.pallas_call(
        functools.partial(_mean_body, inv_w=1.0 / W),
        grid=grid,
        in_specs=[pl.BlockSpec((block_n, W, D), lambda i: (i, 0, 0))],
        out_specs=pl.BlockSpec((block_n, D), lambda i: (i, 0)),
        out_shape=jax.ShapeDtypeStruct((V, D), jnp.float32),
    )(final_emb)


def _make_sc_gather(V, D, B):
    info = plsc.get_sparse_core_info()
    NC, NS = info.num_cores, info.num_subcores
    NW = NC * NS
    b_per_w = B // NW
    CHUNK = 128
    n_chunks = b_per_w // CHUNK
    mesh = plsc.VectorSubcoreMesh(core_axis_name="c", subcore_axis_name="s")

    @functools.partial(
        pl.kernel,
        mesh=mesh,
        out_type=jax.ShapeDtypeStruct((B, D), jnp.float32),
        scratch_types=[
            pltpu.VMEM((n_chunks, CHUNK), jnp.int32),
            pltpu.VMEM((b_per_w, D), jnp.float32),
            pltpu.SemaphoreType.DMA,
            pltpu.SemaphoreType.DMA,
            pltpu.SemaphoreType.DMA,
            pltpu.SemaphoreType.DMA,
            pltpu.SemaphoreType.DMA,
        ],
    )
    def gather(table_hbm, idx_hbm, out_hbm, idx_v, rows_v,
               g0, g1, g2, g3, wsem):
        wid = lax.axis_index("s") * NC + lax.axis_index("c")
        gsems = (g0, g1, g2, g3)
        pltpu.sync_copy(idx_hbm.at[pl.ds(wid * n_chunks, n_chunks)], idx_v)
        gathers = []
        for j in range(n_chunks):
            gathers.append(pltpu.async_copy(
                table_hbm.at[idx_v.at[j]],
                rows_v.at[pl.ds(j * CHUNK, CHUNK)],
                gsems[j],
            ))
        writes = []
        for j in range(n_chunks):
            gathers[j].wait()
            writes.append(pltpu.async_copy(
                rows_v.at[pl.ds(j * CHUNK, CHUNK)],
                out_hbm.at[pl.ds(wid * b_per_w + j * CHUNK, CHUNK)],
                wsem,
            ))
        for wr in writes:
            wr.wait()

    return gather


def kernel(final_emb, node, time):
    V, W, D = final_emb.shape
    B = node.shape[0]
    table = _time_mean(final_emb)
    idx = node.reshape(B // 128, 128).astype(jnp.int32)
    rows = _make_sc_gather(V, D, B)(table, idx)
    return rows.reshape(B, 1, D)
